# baseline (speedup 1.0000x reference)
import functools

import jax
import jax.numpy as jnp
from jax import lax
from jax.experimental import pallas as pl
from jax.experimental.pallas import tpu as pltpu
from jax.experimental.pallas import tpu_sc as plsc

B, D, H, V = 4096, 1024, 2048, 1000
BB = 1024
NC, NS = 2, 16
NW = NC * NS
BPW = B // NW
L = 16


def _tc_body(x_ref, w1_ref, b1_ref, w2_ref, b2_ref, logits_ref, lse_ref,
             w1b_ref, w2b_ref):
    @pl.when(pl.program_id(0) == 0)
    def _cast_weights():
        w1b_ref[...] = w1_ref[...].astype(jnp.bfloat16)
        w2b_ref[...] = w2_ref[...].astype(jnp.bfloat16)

    x = x_ref[...].astype(jnp.bfloat16)
    h = jnp.maximum(
        jnp.dot(x, w1b_ref[...], preferred_element_type=jnp.float32) + b1_ref[...],
        0.0,
    ).astype(jnp.bfloat16)
    logits = jnp.dot(h, w2b_ref[...], preferred_element_type=jnp.float32) + b2_ref[...]
    cols = jax.lax.broadcasted_iota(jnp.int32, logits.shape, 1)
    lv = jnp.where(cols < V, logits, jnp.float32(-1e4))
    s = jnp.sum(jnp.exp(lv), axis=1, keepdims=True)
    logits_ref[...] = logits
    lse_ref[...] = jnp.log(s)


@functools.partial(
    pl.kernel,
    mesh=plsc.VectorSubcoreMesh(core_axis_name="c", subcore_axis_name="s"),
    out_type=jax.ShapeDtypeStruct((B,), jnp.float32),
    scratch_types=[
        pltpu.VMEM((BPW,), jnp.int32),
        pltpu.VMEM((BPW,), jnp.int32),
        pltpu.VMEM((BPW,), jnp.float32),
        pltpu.VMEM((BPW,), jnp.float32),
        pltpu.VMEM((BPW,), jnp.float32),
        pltpu.SemaphoreType.DMA,
    ],
)
def _sc_gather(logits_hbm, idx_hbm, lse_hbm, out_hbm,
               idx_v, fl_v, lse_v, g_v, o_v, sem):
    wid = lax.axis_index("s") * NC + lax.axis_index("c")
    base = wid * BPW
    pltpu.sync_copy(idx_hbm.at[pl.ds(base, BPW)], idx_v)
    pltpu.sync_copy(lse_hbm.at[pl.ds(base, BPW)], lse_v)
    for j in range(BPW // L):
        sl = pl.ds(j * L, L)
        rows = base + j * L + lax.iota(jnp.int32, L)
        fl_v[sl] = rows * V + idx_v[sl]
    pltpu.async_copy(logits_hbm.at[fl_v], g_v, sem).wait()
    for j in range(BPW // L):
        sl = pl.ds(j * L, L)
        o_v[sl] = g_v[sl] - lse_v[sl]
    pltpu.sync_copy(o_v, out_hbm.at[pl.ds(base, BPW)])


def kernel(state, true_samples, W1, b1, W2, b2):
    idx = true_samples.astype(jnp.int32).reshape(B)
    logits, lse = pl.pallas_call(
        _tc_body,
        grid=(B // BB,),
        in_specs=[
            pl.BlockSpec((BB, D), lambda i: (i, 0)),
            pl.BlockSpec((D, H), lambda i: (0, 0)),
            pl.BlockSpec((1, H), lambda i: (0, 0)),
            pl.BlockSpec((H, V), lambda i: (0, 0)),
            pl.BlockSpec((1, V), lambda i: (0, 0)),
        ],
        out_specs=[
            pl.BlockSpec((BB, V), lambda i: (i, 0)),
            pl.BlockSpec((BB, 1), lambda i: (i, 0)),
        ],
        out_shape=[
            jax.ShapeDtypeStruct((B, V), jnp.float32),
            jax.ShapeDtypeStruct((B, 1), jnp.float32),
        ],
        scratch_shapes=[
            pltpu.VMEM((D, H), jnp.bfloat16),
            pltpu.VMEM((H, V), jnp.bfloat16),
        ],
    )(state, W1, b1.reshape(1, H), W2, b2.reshape(1, V))
    g = _sc_gather(logits.reshape(B * V), idx, lse.reshape(B))
    return (true_samples, g.reshape(B, 1))


# speedup vs baseline: 1.5108x; 1.5108x over previous
import jax
import jax.numpy as jnp
from jax.experimental import pallas as pl
from jax.experimental.pallas import tpu as pltpu

B, D, H, V = 4096, 1024, 2048, 1000
BB = 1024


def _tc_body(x_ref, idx_ref, w1_ref, b1_ref, w2_ref, b2_ref, out_ref,
             w1b_ref, w2b_ref):
    @pl.when(pl.program_id(0) == 0)
    def _cast_weights():
        w1b_ref[...] = w1_ref[...].astype(jnp.bfloat16)
        w2b_ref[...] = w2_ref[...].astype(jnp.bfloat16)

    x = x_ref[...].astype(jnp.bfloat16)
    h = jnp.maximum(
        jnp.dot(x, w1b_ref[...], preferred_element_type=jnp.float32) + b1_ref[...],
        0.0,
    ).astype(jnp.bfloat16)
    logits = jnp.dot(h, w2b_ref[...], preferred_element_type=jnp.float32) + b2_ref[...]
    cols = jax.lax.broadcasted_iota(jnp.int32, logits.shape, 1)
    lv = jnp.where(cols < V, logits, jnp.float32(-1e4))
    eb = jnp.exp(lv).astype(jnp.bfloat16)
    gb = jnp.where(cols == idx_ref[...], logits, 0.0).astype(jnp.bfloat16)
    onesv = jnp.ones((eb.shape[1], 1), jnp.bfloat16)
    s = jnp.dot(eb, onesv, preferred_element_type=jnp.float32)
    g = jnp.dot(gb, onesv, preferred_element_type=jnp.float32)
    out_ref[...] = g - jnp.log(s)


def kernel(state, true_samples, W1, b1, W2, b2):
    idx = true_samples.astype(jnp.int32)
    out = pl.pallas_call(
        _tc_body,
        grid=(B // BB,),
        in_specs=[
            pl.BlockSpec((BB, D), lambda i: (i, 0)),
            pl.BlockSpec((BB, 1), lambda i: (i, 0)),
            pl.BlockSpec((D, H), lambda i: (0, 0)),
            pl.BlockSpec((1, H), lambda i: (0, 0)),
            pl.BlockSpec((H, V), lambda i: (0, 0)),
            pl.BlockSpec((1, V), lambda i: (0, 0)),
        ],
        out_specs=pl.BlockSpec((BB, 1), lambda i: (i, 0)),
        out_shape=jax.ShapeDtypeStruct((B, 1), jnp.float32),
        scratch_shapes=[
            pltpu.VMEM((D, H), jnp.bfloat16),
            pltpu.VMEM((H, V), jnp.bfloat16),
        ],
    )(state, idx, W1, b1.reshape(1, H), W2, b2.reshape(1, V))
    return (true_samples, out)


# speedup vs baseline: 1.5737x; 1.0417x over previous
import jax
import jax.numpy as jnp
from jax.experimental import pallas as pl
from jax.experimental.pallas import tpu as pltpu

B, D, H, V = 4096, 1024, 2048, 1000
BB = 1024


def _tc_body(x_ref, idx_ref, w1_ref, b1_ref, w2_ref, b2_ref, out_ref,
             w1b_ref, w2b_ref):
    @pl.when(pl.program_id(0) == 0)
    def _cast_weights():
        w1b_ref[...] = w1_ref[...].astype(jnp.bfloat16)
        w2b_ref[...] = w2_ref[...].astype(jnp.bfloat16)

    HB = BB // 2
    for k in range(2):
        rows = pl.ds(k * HB, HB)
        x = x_ref[rows, :].astype(jnp.bfloat16)
        h = jnp.maximum(
            jnp.dot(x, w1b_ref[...], preferred_element_type=jnp.float32)
            + b1_ref[...],
            0.0,
        ).astype(jnp.bfloat16)
        logits = (jnp.dot(h, w2b_ref[...], preferred_element_type=jnp.float32)
                  + b2_ref[...])
        cols = jax.lax.broadcasted_iota(jnp.int32, logits.shape, 1)
        lv = jnp.where(cols < V, logits, jnp.float32(-1e4))
        s = jnp.sum(jnp.exp(lv), axis=1, keepdims=True)
        g = jnp.sum(jnp.where(cols == idx_ref[rows, :], logits, 0.0),
                    axis=1, keepdims=True)
        out_ref[rows, :] = g - jnp.log(s)


def kernel(state, true_samples, W1, b1, W2, b2):
    idx = true_samples.astype(jnp.int32)
    out = pl.pallas_call(
        _tc_body,
        grid=(B // BB,),
        in_specs=[
            pl.BlockSpec((BB, D), lambda i: (i, 0)),
            pl.BlockSpec((BB, 1), lambda i: (i, 0)),
            pl.BlockSpec((D, H), lambda i: (0, 0)),
            pl.BlockSpec((1, H), lambda i: (0, 0)),
            pl.BlockSpec((H, V), lambda i: (0, 0)),
            pl.BlockSpec((1, V), lambda i: (0, 0)),
        ],
        out_specs=pl.BlockSpec((BB, 1), lambda i: (i, 0)),
        out_shape=jax.ShapeDtypeStruct((B, 1), jnp.float32),
        scratch_shapes=[
            pltpu.VMEM((D, H), jnp.bfloat16),
            pltpu.VMEM((H, V), jnp.bfloat16),
        ],
    )(state, idx, W1, b1.reshape(1, H), W2, b2.reshape(1, V))
    return (true_samples, out)


# speedup vs baseline: 1.5929x; 1.0122x over previous
import jax
import jax.numpy as jnp
from jax.experimental import pallas as pl
from jax.experimental.pallas import tpu as pltpu

B, D, H, V = 4096, 1024, 2048, 1000
BB = 1024


def _tc_body(x_ref, idx_ref, w1_ref, b1_ref, w2_ref, b2_ref, out_ref,
             w1b_ref, w2b_ref):
    @pl.when(pl.program_id(0) == 0)
    def _cast_weights():
        w1b_ref[...] = w1_ref[...].astype(jnp.bfloat16)
        w2b_ref[...] = w2_ref[...].astype(jnp.bfloat16)

    x = x_ref[...].astype(jnp.bfloat16)
    h = jnp.maximum(
        jnp.dot(x, w1b_ref[...], preferred_element_type=jnp.float32) + b1_ref[...],
        0.0,
    ).astype(jnp.bfloat16)
    logits = jnp.dot(h, w2b_ref[...], preferred_element_type=jnp.float32) + b2_ref[...]
    cols = jax.lax.broadcasted_iota(jnp.int32, logits.shape, 1)
    lv = jnp.where(cols < V, logits, jnp.float32(-1e4))
    s = jnp.sum(jnp.exp(lv), axis=1, keepdims=True)
    g = jnp.sum(jnp.where(cols == idx_ref[...], logits, 0.0), axis=1, keepdims=True)
    out_ref[...] = g - jnp.log(s)


def kernel(state, true_samples, W1, b1, W2, b2):
    idx = true_samples.astype(jnp.int32)
    out = pl.pallas_call(
        _tc_body,
        grid=(B // BB,),
        in_specs=[
            pl.BlockSpec((BB, D), lambda i: (i, 0)),
            pl.BlockSpec((BB, 1), lambda i: (i, 0)),
            pl.BlockSpec((D, H), lambda i: (0, 0)),
            pl.BlockSpec((1, H), lambda i: (0, 0)),
            pl.BlockSpec((H, V), lambda i: (0, 0)),
            pl.BlockSpec((1, V), lambda i: (0, 0)),
        ],
        out_specs=pl.BlockSpec((BB, 1), lambda i: (i, 0)),
        out_shape=jax.ShapeDtypeStruct((B, 1), jnp.float32),
        scratch_shapes=[
            pltpu.VMEM((D, H), jnp.bfloat16),
            pltpu.VMEM((H, V), jnp.bfloat16),
        ],
    )(state, idx, W1, b1.reshape(1, H), W2, b2.reshape(1, V))
    return (true_samples, out)
